# Initial kernel scaffold; baseline (speedup 1.0000x reference)
import jax
import jax.numpy as jnp
from jax.experimental import pallas as pl


def kernel(batched_coordinates):
    raise NotImplementedError("write your pallas kernel here")


# speedup vs baseline: 1.3576x; 1.3576x over previous
import functools

import jax
import jax.numpy as jnp
from jax import lax
from jax.experimental import pallas as pl
from jax.experimental.pallas import tpu as pltpu
from jax.experimental.pallas import tpu_sc as plsc

N_NEIGHBORS = 16
NEIGHBOR_LIMIT = 20
N_BINS = 5

_KNN_R = 32
_HIST_R = 512


def _knn_body(ct_ref, c_ref, nd_ref, idx_ref, rsum_ref):
    n = ct_ref.shape[2]
    b = pl.program_id(1)
    xc = ct_ref[0, 0:1, :]
    yc = ct_ref[0, 1:2, :]
    zc = ct_ref[0, 2:3, :]
    xr = c_ref[0, :, 0:1]
    yr = c_ref[0, :, 1:2]
    zr = c_ref[0, :, 2:3]
    dx = xr - xc
    dy = yr - yc
    dz = zr - zc
    d2 = dx * dx + dy * dy + dz * dz
    work = jnp.sqrt(d2 + 1e-12)
    iota = lax.broadcasted_iota(jnp.int32, work.shape, 1)
    s17 = None
    for k in range(NEIGHBOR_LIMIT):
        m = jnp.min(work, axis=1, keepdims=True)
        eq = work == m
        cand = jnp.where(eq, iota, n)
        am = jnp.min(cand, axis=1, keepdims=True)
        nd_ref[0, :, k] = m[:, 0]
        idx_ref[0, :, k] = am[:, 0]
        if k == N_NEIGHBORS:
            s17 = jnp.sum(m) * (1.0 / n)
        if k < NEIGHBOR_LIMIT - 1:
            work = jnp.where(iota == am, jnp.float32(1e30), work)
    part = jnp.full((1, 1), s17, dtype=jnp.float32)
    prev = jnp.where(b == 0, jnp.zeros_like(part), rsum_ref[0])
    rsum_ref[0] = prev + part


def _knn_call(coords):
    bsz, n, _ = coords.shape
    ct = jnp.transpose(coords, (0, 2, 1))
    grid = (bsz, n // _KNN_R)
    return pl.pallas_call(
        _knn_body,
        grid=grid,
        in_specs=[
            pl.BlockSpec((1, 3, n), lambda s, b: (s, 0, 0)),
            pl.BlockSpec((1, _KNN_R, 3), lambda s, b: (s, b, 0)),
        ],
        out_specs=[
            pl.BlockSpec((1, _KNN_R, NEIGHBOR_LIMIT), lambda s, b: (s, b, 0)),
            pl.BlockSpec((1, _KNN_R, NEIGHBOR_LIMIT), lambda s, b: (s, b, 0)),
            pl.BlockSpec((1, 1, 1), lambda s, b: (s, 0, 0)),
        ],
        out_shape=[
            jax.ShapeDtypeStruct((bsz, n, NEIGHBOR_LIMIT), jnp.float32),
            jax.ShapeDtypeStruct((bsz, n, NEIGHBOR_LIMIT), jnp.int32),
            jax.ShapeDtypeStruct((bsz, 1, 1), jnp.float32),
        ],
    )(ct, coords)


def _sc_gather(cflat, iflat):
    bsz, e = iflat.shape
    nc3 = cflat.shape[1]
    info = plsc.get_sparse_core_info()
    ncores = info.num_cores
    nw = ncores * info.num_subcores
    epw = e // nw
    mesh = plsc.VectorSubcoreMesh(core_axis_name="c", subcore_axis_name="s")

    @functools.partial(
        pl.kernel,
        mesh=mesh,
        compiler_params=pltpu.CompilerParams(needs_layout_passes=False),
        out_type=[jax.ShapeDtypeStruct((bsz, e), jnp.float32)] * 3,
        scratch_types=[
            pltpu.VMEM((nc3,), jnp.float32),
            pltpu.VMEM((epw,), jnp.int32),
            pltpu.VMEM((epw,), jnp.float32),
            pltpu.VMEM((epw,), jnp.float32),
            pltpu.VMEM((epw,), jnp.float32),
        ],
    )
    def gk(c_hbm, i_hbm, ox_hbm, oy_hbm, oz_hbm, cv, iv, bx, by, bz):
        wid = lax.axis_index("s") * ncores + lax.axis_index("c")
        base = wid * epw
        for s in range(bsz):
            pltpu.sync_copy(c_hbm.at[s], cv)
            pltpu.sync_copy(i_hbm.at[s, pl.ds(base, epw)], iv)

            def body(i, _):
                off = i * 16
                ii = iv[pl.ds(off, 16)] * 3
                bx[pl.ds(off, 16)] = plsc.load_gather(cv, [ii])
                by[pl.ds(off, 16)] = plsc.load_gather(cv, [ii + 1])
                bz[pl.ds(off, 16)] = plsc.load_gather(cv, [ii + 2])
                return 0

            lax.fori_loop(0, epw // 16, body, 0)
            pltpu.sync_copy(bx, ox_hbm.at[s, pl.ds(base, epw)])
            pltpu.sync_copy(by, oy_hbm.at[s, pl.ds(base, epw)])
            pltpu.sync_copy(bz, oz_hbm.at[s, pl.ds(base, epw)])

    return gk(cflat, iflat)


def _cross(ax, ay, az, bx, by, bz):
    return (ay * bz - az * by, az * bx - ax * bz, ax * by - ay * bx)


def _rp(x):
    u = jax.lax.bitcast_convert_type(x, jnp.int32)
    r = u + jnp.int32(0x7FFF) + jnp.bitwise_and(
        jax.lax.shift_right_arithmetic(u, jnp.int32(16)), jnp.int32(1))
    r = jnp.bitwise_and(r, jnp.int32(-65536))
    return jax.lax.bitcast_convert_type(r, jnp.float32)


def _neigh_parts(nd_ref, gx_ref, gy_ref, gz_ref, c_ref, radius):
    nd = nd_ref[0]
    cx = c_ref[0, :, 0:1]
    cy = c_ref[0, :, 1:2]
    cz = c_ref[0, :, 2:3]
    mask = nd <= radius
    fl = 2.0 * radius
    nx = jnp.where(mask, gx_ref[0] - cx, fl)
    ny = jnp.where(mask, gy_ref[0] - cy, fl)
    nz = jnp.where(mask, gz_ref[0] - cz, fl)
    return nx, ny, nz


def _cov_body(nd_ref, gx_ref, gy_ref, gz_ref, c_ref, rs_ref, cov_ref):
    radius = rs_ref[0]
    nx, ny, nz = _neigh_parts(nd_ref, gx_ref, gy_ref, gz_ref, c_ref, radius)
    d = jnp.sqrt(nx * nx + ny * ny + nz * nz + 1e-12)
    w = jnp.maximum(radius - d, 0.0)
    denom = jnp.sum(w, axis=1, keepdims=True) + 1e-12
    wnx = _rp(w * nx)
    wny = _rp(w * ny)
    wnz = _rp(w * nz)
    rnx = _rp(nx)
    rny = _rp(ny)
    rnz = _rp(nz)

    def s(p, q):
        return jnp.sum(p * q, axis=1, keepdims=True) / denom

    a = s(wnx, rnx)
    b = s(wny, rny)
    c = s(wnz, rnz)
    dxy = (s(wnx, rny) + s(wny, rnx)) * 0.5
    dxz = (s(wnx, rnz) + s(wnz, rnx)) * 0.5
    dyz = (s(wny, rnz) + s(wnz, rny)) * 0.5
    zero = jnp.zeros_like(a)
    cov_ref[0] = jnp.concatenate([a, b, c, dxy, dxz, dyz, zero, zero], axis=1)


def _cov_call(nd, gx, gy, gz, coords, rsum):
    bsz, n, _ = coords.shape
    grid = (bsz, n // _HIST_R)
    blk20 = pl.BlockSpec((1, _HIST_R, NEIGHBOR_LIMIT), lambda s, b: (s, b, 0))
    return pl.pallas_call(
        _cov_body,
        grid=grid,
        in_specs=[
            blk20, blk20, blk20, blk20,
            pl.BlockSpec((1, _HIST_R, 3), lambda s, b: (s, b, 0)),
            pl.BlockSpec((1, 1, 1), lambda s, b: (s, 0, 0)),
        ],
        out_specs=pl.BlockSpec((1, _HIST_R, 8), lambda s, b: (s, b, 0)),
        out_shape=jax.ShapeDtypeStruct((bsz, n, 8), jnp.float32),
    )(nd, gx, gy, gz, coords, rsum)


def _eigvec(lam, a, b, c, dd, ee, ff, fbx, fby, fbz):
    m00 = a - lam
    m11 = b - lam
    m22 = c - lam
    c01 = _cross(m00, dd, ee, dd, m11, ff)
    c12 = _cross(dd, m11, ff, ee, ff, m22)
    c20 = _cross(ee, ff, m22, m00, dd, ee)
    n01 = c01[0] * c01[0] + c01[1] * c01[1] + c01[2] * c01[2]
    n12 = c12[0] * c12[0] + c12[1] * c12[1] + c12[2] * c12[2]
    n20 = c20[0] * c20[0] + c20[1] * c20[1] + c20[2] * c20[2]
    use12 = jnp.logical_and(n12 >= n01, n12 >= n20)
    use01 = jnp.logical_and(jnp.logical_not(use12), n01 >= n20)
    vx = jnp.where(use12, c12[0], jnp.where(use01, c01[0], c20[0]))
    vy = jnp.where(use12, c12[1], jnp.where(use01, c01[1], c20[1]))
    vz = jnp.where(use12, c12[2], jnp.where(use01, c01[2], c20[2]))
    nn = vx * vx + vy * vy + vz * vz
    good = nn > 1e-30
    inv = lax.rsqrt(jnp.where(good, nn, 1.0))
    vx = jnp.where(good, vx * inv, fbx)
    vy = jnp.where(good, vy * inv, fby)
    vz = jnp.where(good, vz * inv, fbz)
    return vx, vy, vz


def _hist_body(nd_ref, gx_ref, gy_ref, gz_ref, c_ref, rs_ref, xa_ref, za_ref,
               out_ref):
    radius = rs_ref[0]
    nx, ny, nz = _neigh_parts(nd_ref, gx_ref, gy_ref, gz_ref, c_ref, radius)
    xax = xa_ref[0, :, 0:1]
    xay = xa_ref[0, :, 1:2]
    xaz = xa_ref[0, :, 2:3]
    zax = za_ref[0, :, 0:1]
    zay = za_ref[0, :, 1:2]
    zaz = za_ref[0, :, 2:3]

    def disamb(ux, uy, uz):
        s = jnp.sum(nx * ux + ny * uy + nz * uz, axis=1, keepdims=True)
        s = jnp.sign(s)
        s = jnp.where(s == 0, 1.0, s)
        return ux * s, uy * s, uz * s

    xax, xay, xaz = disamb(xax, xay, xaz)
    zax, zay, zaz = disamb(zax, zay, zaz)
    yax, yay, yaz = _cross(zax, zay, zaz, xax, xay, xaz)
    rnx, rny, rnz = _rp(nx), _rp(ny), _rp(nz)
    lx = _rp(xax) * rnx + _rp(xay) * rny + _rp(xaz) * rnz
    ly = _rp(yax) * rnx + _rp(yay) * rny + _rp(yaz) * rnz
    lz = _rp(zax) * rnx + _rp(zay) * rny + _rp(zaz) * rnz
    r3 = jnp.sqrt(lx * lx + ly * ly + lz * lz + 1e-12)
    r2 = jnp.sqrt(lx * lx + ly * ly + 1e-12)
    sc = r3 / (r2 + 1e-12)
    px = lx * sc
    py = ly * sc
    step = 2.0 * radius * (1.0 / N_BINS)
    bxf = jnp.floor((px + radius) / step)
    byf = jnp.floor((py + radius) / step)
    valid = ((bxf >= 0) & (bxf < N_BINS) & (byf >= 0) & (byf < N_BINS))
    vm = valid.astype(jnp.float32)
    bx = jnp.clip(bxf, 0, N_BINS - 1).astype(jnp.int32)
    by = jnp.clip(byf, 0, N_BINS - 1).astype(jnp.int32)
    bix = by * N_BINS + bx
    cols = []
    ssq = None
    for j in range(N_BINS * N_BINS):
        jy, jx = divmod(j, N_BINS)
        cpx = -radius + jx * step + step * 0.5
        cpy = -radius + jy * step + step * 0.5
        wt = radius - jnp.sqrt(cpx * cpx + cpy * cpy + 1e-12)
        h = jnp.sum(vm * (bix == j).astype(jnp.float32), axis=1,
                    keepdims=True) * wt
        cols.append(h)
        ssq = h * h if ssq is None else ssq + h * h
    hnorm = 1.0 / jnp.sqrt(ssq + 1e-12)
    out_ref[0] = jnp.concatenate(cols, axis=1) * hnorm


def _hist_call(nd, gx, gy, gz, coords, rsum, xa, za):
    bsz, n, _ = coords.shape
    grid = (bsz, n // _HIST_R)
    blk20 = pl.BlockSpec((1, _HIST_R, NEIGHBOR_LIMIT), lambda s, b: (s, b, 0))
    blk3 = pl.BlockSpec((1, _HIST_R, 3), lambda s, b: (s, b, 0))
    return pl.pallas_call(
        _hist_body,
        grid=grid,
        in_specs=[
            blk20, blk20, blk20, blk20, blk3,
            pl.BlockSpec((1, 1, 1), lambda s, b: (s, 0, 0)),
            blk3, blk3,
        ],
        out_specs=pl.BlockSpec((1, _HIST_R, N_BINS * N_BINS),
                               lambda s, b: (s, b, 0)),
        out_shape=jax.ShapeDtypeStruct((bsz, n, N_BINS * N_BINS),
                                       jnp.float32),
    )(nd, gx, gy, gz, coords, rsum, xa, za)


def _eigh_axes(cov8):
    a, b, c = cov8[..., 0], cov8[..., 1], cov8[..., 2]
    dxy, dxz, dyz = cov8[..., 3], cov8[..., 4], cov8[..., 5]
    r0 = jnp.stack([a, dxy, dxz], axis=-1)
    r1 = jnp.stack([dxy, b, dyz], axis=-1)
    r2 = jnp.stack([dxz, dyz, c], axis=-1)
    cov = jnp.stack([r0, r1, r2], axis=-2)

    def f(m):
        evals, evecs = jnp.linalg.eigh(m)
        return evecs[:, :, 2], evecs[:, :, 0]

    return jax.lax.map(f, cov)


def kernel(batched_coordinates):
    coords = batched_coordinates
    bsz, n, _ = coords.shape
    nd, idx, rsum = _knn_call(coords)
    cflat = coords.reshape(bsz, n * 3)
    iflat = idx.reshape(bsz, n * NEIGHBOR_LIMIT)
    gx, gy, gz = _sc_gather(cflat, iflat)
    shp = (bsz, n, NEIGHBOR_LIMIT)
    gx, gy, gz = gx.reshape(shp), gy.reshape(shp), gz.reshape(shp)
    del rsum
    rsum = jnp.mean(nd[:, :, N_NEIGHBORS], axis=1).reshape(bsz, 1, 1)
    cov8 = _cov_call(nd, gx, gy, gz, coords, rsum)
    xa, za = _eigh_axes(cov8)
    return _hist_call(nd, gx, gy, gz, coords, rsum, xa, za)
